# baseline (speedup 1.0000x reference)
import functools

import jax
import jax.numpy as jnp
from jax import lax
from jax.experimental import pallas as pl
from jax.experimental.pallas import tpu as pltpu
from jax.experimental.pallas import tpu_sc as plsc

_EMBED = 64
_BATCH = 16384
_NROWS = 1000000
_REG_LAMBDA = 0.0001

_NC = 2
_NS = 16
_NW = _NC * _NS
_BPW = _BATCH // _NW
_PH = _BPW // 2

_RB = 512
_RGRID = -(-_NROWS // _RB)
_NLINES = _RGRID * 256


def _repack_body(tin_ref, out_ref):
    out_ref[:, 0:_EMBED] = jnp.transpose(tin_ref[:, 0:256])
    out_ref[:, _EMBED:128] = jnp.transpose(tin_ref[:, 256:512])


_repack = pl.pallas_call(
    _repack_body,
    grid=(_RGRID,),
    in_specs=[pl.BlockSpec((_EMBED, _RB), lambda j: (0, j))],
    out_specs=pl.BlockSpec((256, 128), lambda j: (j, 0)),
    out_shape=jax.ShapeDtypeStruct((_NLINES, 128), jnp.float32),
)


@functools.partial(
    pl.kernel,
    mesh=plsc.VectorSubcoreMesh(core_axis_name="c", subcore_axis_name="s"),
    compiler_params=pltpu.CompilerParams(needs_layout_passes=False),
    out_type=(
        jax.ShapeDtypeStruct((128, 128), jnp.float32),
        jax.ShapeDtypeStruct((_NW, 16), jnp.float32),
    ),
    scratch_types=[
        pltpu.VMEM((4, 128), jnp.int32),
        pltpu.VMEM((4, 128), jnp.int32),
        pltpu.VMEM((4, 128), jnp.int32),
        pltpu.VMEM((4, 128), jnp.int32),
        pltpu.VMEM((4, 128), jnp.int32),
        pltpu.VMEM((4, 128), jnp.int32),
        pltpu.VMEM((_PH, 128), jnp.float32),
        pltpu.VMEM((_PH, 128), jnp.float32),
        pltpu.VMEM((_PH, 128), jnp.float32),
        pltpu.VMEM((4, 128), jnp.float32),
        pltpu.VMEM((16,), jnp.float32),
        pltpu.SemaphoreType.DMA,
    ],
)
def _sc_scores(user_hbm, pos_hbm, neg_hbm, utab_hbm, itab_hbm,
               s_out, reg_out,
               idx_u, idx_p, idx_n, lin_u, lin_p, lin_n,
               rows_u, rows_p, rows_n, s_loc, ssq_loc, sem):
    wid = lax.axis_index("s") * _NC + lax.axis_index("c")
    r0 = wid * 4
    pltpu.sync_copy(user_hbm.at[pl.ds(r0, 4)], idx_u)
    pltpu.sync_copy(pos_hbm.at[pl.ds(r0, 4)], idx_p)
    pltpu.sync_copy(neg_hbm.at[pl.ds(r0, 4)], idx_n)

    for src, dst in ((idx_u, lin_u), (idx_p, lin_p), (idx_n, lin_n)):
        for a in range(4):
            for b in range(8):
                sl = pl.ds(b * 16, 16)
                r = src[a, sl]
                dst[a, sl] = lax.bitwise_or(
                    lax.shift_left(lax.shift_right_logical(r, 9), 8),
                    lax.bitwise_and(r, 255))

    zero = jnp.zeros((16,), jnp.float32)
    sq0 = zero
    sq1 = zero

    for ph in range(2):
        copies = []
        for c in range(2):
            k = ph * 2 + c
            dst = pl.ds(c * 128, 128)
            copies.append(pltpu.async_copy(
                utab_hbm.at[lin_u.at[k]], rows_u.at[dst], sem))
            copies.append(pltpu.async_copy(
                itab_hbm.at[lin_p.at[k]], rows_p.at[dst], sem))
            copies.append(pltpu.async_copy(
                itab_hbm.at[lin_n.at[k]], rows_n.at[dst], sem))
        for cp in copies:
            cp.wait()

        def group(g, sq_carry):
            q0, q1 = sq_carry
            gg = ph * 16 + g
            a, b = gg >> 3, (gg & 7) * 16
            lane = lax.iota(jnp.int32, 16)
            base = (lane + g * 16) * 128
            def _off(r):
                return lax.bitwise_and(lax.shift_right_logical(r, 8), 1) * _EMBED

            bu = base + _off(idx_u[a, pl.ds(b, 16)])
            bp = base + _off(idx_p[a, pl.ds(b, 16)])
            bn = base + _off(idx_n[a, pl.ds(b, 16)])

            def dstep(dv, carry):
                acc0, acc1, c0, c1 = carry
                d0 = dv * 4
                for j in range(4):
                    fu = bu + (d0 + j)
                    fp = bp + (d0 + j)
                    fn = bn + (d0 + j)
                    cu = plsc.load_gather(
                        rows_u, [lax.shift_right_logical(fu, 7),
                                 lax.bitwise_and(fu, 127)])
                    cp = plsc.load_gather(
                        rows_p, [lax.shift_right_logical(fp, 7),
                                 lax.bitwise_and(fp, 127)])
                    cn = plsc.load_gather(
                        rows_n, [lax.shift_right_logical(fn, 7),
                                 lax.bitwise_and(fn, 127)])
                    prod = cu * (cp - cn)
                    sq = cu * cu + (cp * cp + cn * cn)
                    if j % 2 == 0:
                        acc0 = acc0 + prod
                        c0 = c0 + sq
                    else:
                        acc1 = acc1 + prod
                        c1 = c1 + sq
                return acc0, acc1, c0, c1

            acc0, acc1, q0, q1 = lax.fori_loop(
                0, _EMBED // 4, dstep, (zero, zero, q0, q1))
            s_loc[gg >> 3, pl.ds((gg & 7) * 16, 16)] = acc0 + acc1
            return q0, q1

        sq0, sq1 = lax.fori_loop(0, 16, group, (sq0, sq1))

    ssq_loc[...] = sq0 + sq1
    pltpu.sync_copy(s_loc, s_out.at[pl.ds(wid * 4, 4)])
    pltpu.sync_copy(ssq_loc, reg_out.at[wid])


def _tc_finish(s_ref, regs_ref, bpr_ref, reg_ref):
    s = s_ref[...]
    sp = jnp.maximum(-s, 0.0) + jnp.log1p(jnp.exp(-jnp.abs(s)))
    bpr_ref[0, 0] = jnp.sum(sp) * (1.0 / _BATCH)
    reg_ref[0, 0] = jnp.sum(regs_ref[...]) * (_REG_LAMBDA / (2.0 * _BATCH))


_tc_fin = pl.pallas_call(
    _tc_finish,
    out_shape=(
        jax.ShapeDtypeStruct((1, 1), jnp.float32),
        jax.ShapeDtypeStruct((1, 1), jnp.float32),
    ),
    in_specs=[
        pl.BlockSpec(memory_space=pltpu.VMEM),
        pl.BlockSpec(memory_space=pltpu.VMEM),
    ],
    out_specs=(
        pl.BlockSpec(memory_space=pltpu.SMEM),
        pl.BlockSpec(memory_space=pltpu.SMEM),
    ),
)


def kernel(user, positive, negative, user_table, item_table):
    u2 = user.reshape(_BATCH // 128, 128)
    p2 = positive.reshape(_BATCH // 128, 128)
    n2 = negative.reshape(_BATCH // 128, 128)
    ut = _repack(user_table.T)
    it = _repack(item_table.T)
    s, regs = _sc_scores(u2, p2, n2, ut, it)
    bpr, reg = _tc_fin(s, regs.reshape(4, 128))
    return (bpr[0, 0], reg[0, 0])
